# baseline (speedup 1.0000x reference)
import functools

import jax
import jax.numpy as jnp
from jax import lax
from jax.experimental import pallas as pl
from jax.experimental.pallas import tpu as pltpu
from jax.experimental.pallas import tpu_sc as plsc

NUM_NODES = 1000000
RANK = 64
CHUNKS = 2
BATCH = 16384
D = RANK * CHUNKS

HALVES = 2
BH = BATCH // HALVES

NC = 2
NS = 16
NW = NC * NS
B_PER_W = BH // NW
GCHUNK = 128
NCHUNK = B_PER_W // GCHUNK


def _sc_gather_body(idx_hbm, table_hbm, mask_hbm, rows_out, mask_out,
                    idx_v, rows_v, mval_v, sems, sem_mask, sem_out):
    wid = lax.axis_index("s") * NC + lax.axis_index("c")
    base = wid * B_PER_W
    pltpu.sync_copy(idx_hbm.at[pl.ds(base, B_PER_W)], idx_v)
    gathers = []
    for j in range(NCHUNK):
        sl = pl.ds(j * GCHUNK, GCHUNK)
        gathers.append(pltpu.async_copy(
            table_hbm.at[idx_v.at[sl]], rows_v.at[sl, :], sems.at[j]))
    mask_copies = [
        pltpu.async_copy(mask_hbm.at[idx_v.at[pl.ds(j * GCHUNK, GCHUNK)]],
                         mval_v.at[pl.ds(j * GCHUNK, GCHUNK)], sem_mask)
        for j in range(NCHUNK)
    ]
    writes = []
    for j in range(NCHUNK):
        sl = pl.ds(j * GCHUNK, GCHUNK)
        gathers[j].wait()
        writes.append(pltpu.async_copy(
            rows_v.at[sl, :], rows_out.at[pl.ds(base + j * GCHUNK, GCHUNK)],
            sem_out))
    for c in mask_copies:
        c.wait()
    pltpu.sync_copy(mval_v, mask_out.at[pl.ds(base, B_PER_W)])
    for c in writes:
        c.wait()


def _sc_gather(idx_half, table, leaf_mask):
    mesh = plsc.VectorSubcoreMesh(core_axis_name="c", subcore_axis_name="s")
    k = pl.kernel(
        _sc_gather_body,
        out_type=(
            jax.ShapeDtypeStruct((BH, D), jnp.float32),
            jax.ShapeDtypeStruct((BH,), jnp.int32),
        ),
        mesh=mesh,
        scratch_types=[
            pltpu.VMEM((B_PER_W,), jnp.int32),
            pltpu.VMEM((B_PER_W, D), jnp.float32),
            pltpu.VMEM((B_PER_W,), jnp.int32),
            pltpu.SemaphoreType.DMA((NCHUNK,)),
            pltpu.SemaphoreType.DMA,
            pltpu.SemaphoreType.DMA,
        ],
    )
    return k(idx_half, table, leaf_mask)


ROWS_BLK = 2048
BLKS_H = BH // ROWS_BLK

_DN_RT = (((0,), (1,)), ((), ()))
_DN_LT = (((0,), (0,)), ((), ()))


def _tc_mlp_body(x_ref, m_ref, w10_ref, b10_ref, w20_ref, b20_ref,
                 w11_ref, b11_ref, w21_ref, b21_ref, *rest):
    o0_ref, o1_ref = rest[-2], rest[-1]
    x = x_ref[...]
    nl = (m_ref[...] == 0)[None, :]
    e0 = x[:, :RANK]
    e1 = x[:, RANK:]
    eye = (lax.broadcasted_iota(jnp.int32, (RANK, RANK), 0)
           == lax.broadcasted_iota(jnp.int32, (RANK, RANK), 1)
           ).astype(jnp.float32)
    e0t = lax.dot_general(eye, e0, _DN_RT,
                          preferred_element_type=jnp.float32)
    e1t = lax.dot_general(eye, e1, _DN_RT,
                          preferred_element_type=jnp.float32)
    h0t = jnp.maximum(
        lax.dot_general(w10_ref[...], e0, _DN_RT,
                        preferred_element_type=jnp.float32) + b10_ref[...],
        0.0)
    o0t = lax.dot_general(w20_ref[...], h0t, _DN_LT,
                          preferred_element_type=jnp.float32) + b20_ref[...]
    h1t = jnp.maximum(
        lax.dot_general(w11_ref[...], e1, _DN_RT,
                        preferred_element_type=jnp.float32) + b11_ref[...],
        0.0)
    o1t = lax.dot_general(w21_ref[...], h1t, _DN_LT,
                          preferred_element_type=jnp.float32) + b21_ref[...]
    o0_ref[...] = jnp.where(nl, o0t, e0t)
    o1_ref[...] = jnp.where(nl, o1t, e1t)


def _tc_mlp_half(half, rows, mvals, weights, partial):
    wspec = pl.BlockSpec((RANK, RANK), lambda i: (0, 0))
    bspec = pl.BlockSpec((RANK, 1), lambda i: (0, 0))
    in_specs = [
        pl.BlockSpec((ROWS_BLK, D), lambda i: (i, 0)),
        pl.BlockSpec((ROWS_BLK,), lambda i: (i,)),
        wspec, bspec, wspec, bspec,
        wspec, bspec, wspec, bspec,
    ]
    operands = [rows, mvals, *weights]
    kwargs = {}
    if partial is not None:
        in_specs += [pl.BlockSpec(memory_space=pl.ANY),
                     pl.BlockSpec(memory_space=pl.ANY)]
        operands += list(partial)
        kwargs["input_output_aliases"] = {10: 0, 11: 1}
    off = half * BLKS_H
    return pl.pallas_call(
        _tc_mlp_body,
        grid=(BLKS_H,),
        in_specs=in_specs,
        out_specs=[
            pl.BlockSpec((RANK, ROWS_BLK), lambda i: (0, off + i)),
            pl.BlockSpec((RANK, ROWS_BLK), lambda i: (0, off + i)),
        ],
        out_shape=[
            jax.ShapeDtypeStruct((RANK, BATCH), jnp.float32),
            jax.ShapeDtypeStruct((RANK, BATCH), jnp.float32),
        ],
        **kwargs,
    )(*operands)


def kernel(nodeIdx, leaf_mask, table,
           W1_0, b1_0, W2_0, b2_0, W1_1, b1_1, W2_1, b2_1):
    idx = nodeIdx.astype(jnp.int32)
    lmask = leaf_mask.astype(jnp.int32)
    weights = (W1_0, b1_0.reshape(RANK, 1), W2_0, b2_0.reshape(RANK, 1),
               W1_1, b1_1.reshape(RANK, 1), W2_1, b2_1.reshape(RANK, 1))
    halves = [
        _sc_gather(lax.slice(idx, (h * BH,), ((h + 1) * BH,)), table, lmask)
        for h in range(HALVES)
    ]
    partial = None
    for h in range(HALVES):
        rows, mvals = halves[h]
        partial = _tc_mlp_half(h, rows, mvals, weights, partial)
    o0t, o1t = partial
    return (o0t.T, o1t.T)


# speedup vs baseline: 1.0073x; 1.0073x over previous
import functools

import jax
import jax.numpy as jnp
from jax import lax
from jax.experimental import pallas as pl
from jax.experimental.pallas import tpu as pltpu
from jax.experimental.pallas import tpu_sc as plsc

NUM_NODES = 1000000
RANK = 64
CHUNKS = 2
BATCH = 16384
D = RANK * CHUNKS

HALVES = 2
BH = BATCH // HALVES

NC = 2
NS = 16
NW = NC * NS
B_PER_W = BH // NW
GCHUNK = 128
NCHUNK = B_PER_W // GCHUNK


def _sc_gather_body(half_off, idx_hbm, table_hbm, mask_hbm, rows_out,
                    mask_out, idx_v, rows_v, mval_v, sems, sem_mask,
                    sem_out):
    wid = lax.axis_index("s") * NC + lax.axis_index("c")
    base = wid * B_PER_W
    pltpu.sync_copy(idx_hbm.at[pl.ds(half_off + base, B_PER_W)], idx_v)
    gathers = []
    for j in range(NCHUNK):
        sl = pl.ds(j * GCHUNK, GCHUNK)
        gathers.append(pltpu.async_copy(
            table_hbm.at[idx_v.at[sl]], rows_v.at[sl, :], sems.at[j]))
    mask_copies = [
        pltpu.async_copy(mask_hbm.at[idx_v.at[pl.ds(j * GCHUNK, GCHUNK)]],
                         mval_v.at[pl.ds(j * GCHUNK, GCHUNK)], sem_mask)
        for j in range(NCHUNK)
    ]
    writes = []
    for j in range(NCHUNK):
        sl = pl.ds(j * GCHUNK, GCHUNK)
        gathers[j].wait()
        writes.append(pltpu.async_copy(
            rows_v.at[sl, :], rows_out.at[pl.ds(base + j * GCHUNK, GCHUNK)],
            sem_out))
    for c in mask_copies:
        c.wait()
    pltpu.sync_copy(mval_v, mask_out.at[pl.ds(base, B_PER_W)])
    for c in writes:
        c.wait()


def _sc_gather(half, idx, table, leaf_mask):
    mesh = plsc.VectorSubcoreMesh(core_axis_name="c", subcore_axis_name="s")
    k = pl.kernel(
        functools.partial(_sc_gather_body, half * BH),
        out_type=(
            jax.ShapeDtypeStruct((BH, D), jnp.float32),
            jax.ShapeDtypeStruct((BH,), jnp.int32),
        ),
        mesh=mesh,
        scratch_types=[
            pltpu.VMEM((B_PER_W,), jnp.int32),
            pltpu.VMEM((B_PER_W, D), jnp.float32),
            pltpu.VMEM((B_PER_W,), jnp.int32),
            pltpu.SemaphoreType.DMA((NCHUNK,)),
            pltpu.SemaphoreType.DMA,
            pltpu.SemaphoreType.DMA,
        ],
    )
    return k(idx, table, leaf_mask)


ROWS_BLK = 2048
BLKS_H = BH // ROWS_BLK

_DN_RT = (((0,), (1,)), ((), ()))
_DN_LT = (((0,), (0,)), ((), ()))


def _tc_mlp_body(x_ref, m_ref, w10_ref, b10_ref, w20_ref, b20_ref,
                 w11_ref, b11_ref, w21_ref, b21_ref, *rest):
    o0_ref, o1_ref = rest[-2], rest[-1]
    x = x_ref[...]
    nl = (m_ref[...] == 0)[None, :]
    e0 = x[:, :RANK]
    e1 = x[:, RANK:]
    eye = (lax.broadcasted_iota(jnp.int32, (RANK, RANK), 0)
           == lax.broadcasted_iota(jnp.int32, (RANK, RANK), 1)
           ).astype(jnp.float32)
    e0t = lax.dot_general(eye, e0, _DN_RT,
                          preferred_element_type=jnp.float32)
    e1t = lax.dot_general(eye, e1, _DN_RT,
                          preferred_element_type=jnp.float32)
    h0t = jnp.maximum(
        lax.dot_general(w10_ref[...], e0, _DN_RT,
                        preferred_element_type=jnp.float32) + b10_ref[...],
        0.0)
    o0t = lax.dot_general(w20_ref[...], h0t, _DN_LT,
                          preferred_element_type=jnp.float32) + b20_ref[...]
    h1t = jnp.maximum(
        lax.dot_general(w11_ref[...], e1, _DN_RT,
                        preferred_element_type=jnp.float32) + b11_ref[...],
        0.0)
    o1t = lax.dot_general(w21_ref[...], h1t, _DN_LT,
                          preferred_element_type=jnp.float32) + b21_ref[...]
    o0_ref[...] = jnp.where(nl, o0t, e0t)
    o1_ref[...] = jnp.where(nl, o1t, e1t)


def _tc_mlp_half(half, rows, mvals, weights, partial):
    wspec = pl.BlockSpec((RANK, RANK), lambda i: (0, 0))
    bspec = pl.BlockSpec((RANK, 1), lambda i: (0, 0))
    in_specs = [
        pl.BlockSpec((ROWS_BLK, D), lambda i: (i, 0)),
        pl.BlockSpec((ROWS_BLK,), lambda i: (i,)),
        wspec, bspec, wspec, bspec,
        wspec, bspec, wspec, bspec,
    ]
    operands = [rows, mvals, *weights]
    kwargs = {}
    if partial is not None:
        in_specs += [pl.BlockSpec(memory_space=pl.ANY),
                     pl.BlockSpec(memory_space=pl.ANY)]
        operands += list(partial)
        kwargs["input_output_aliases"] = {10: 0, 11: 1}
    off = half * BLKS_H
    return pl.pallas_call(
        _tc_mlp_body,
        grid=(BLKS_H,),
        in_specs=in_specs,
        out_specs=[
            pl.BlockSpec((RANK, ROWS_BLK), lambda i: (0, off + i)),
            pl.BlockSpec((RANK, ROWS_BLK), lambda i: (0, off + i)),
        ],
        out_shape=[
            jax.ShapeDtypeStruct((RANK, BATCH), jnp.float32),
            jax.ShapeDtypeStruct((RANK, BATCH), jnp.float32),
        ],
        **kwargs,
    )(*operands)


def kernel(nodeIdx, leaf_mask, table,
           W1_0, b1_0, W2_0, b2_0, W1_1, b1_1, W2_1, b2_1):
    idx = nodeIdx.astype(jnp.int32)
    lmask = leaf_mask.astype(jnp.int32)
    weights = (W1_0, b1_0.reshape(RANK, 1), W2_0, b2_0.reshape(RANK, 1),
               W1_1, b1_1.reshape(RANK, 1), W2_1, b2_1.reshape(RANK, 1))
    halves = [_sc_gather(h, idx, table, lmask) for h in range(HALVES)]
    partial = None
    for h in range(HALVES):
        rows, mvals = halves[h]
        partial = _tc_mlp_half(h, rows, mvals, weights, partial)
    o0t, o1t = partial
    return (o0t.T, o1t.T)


# speedup vs baseline: 1.0529x; 1.0453x over previous
import functools

import jax
import jax.numpy as jnp
from jax import lax
from jax.experimental import pallas as pl
from jax.experimental.pallas import tpu as pltpu
from jax.experimental.pallas import tpu_sc as plsc

NUM_NODES = 1000000
RANK = 64
CHUNKS = 2
BATCH = 16384
D = RANK * CHUNKS

NC = 2
NS = 16
NW = NC * NS
B_PER_W = BATCH // NW
GCHUNK = 128
NCHUNK = B_PER_W // GCHUNK


def _sc_gather_body(idx_hbm, table_hbm, mask_hbm, rows_out, mask_out,
                    idx_v, rows_v, mval_v, sems, sem_mask, sem_out):
    wid = lax.axis_index("s") * NC + lax.axis_index("c")
    base = wid * B_PER_W
    pltpu.sync_copy(idx_hbm.at[pl.ds(base, B_PER_W)], idx_v)
    gathers = []
    for j in range(NCHUNK):
        sl = pl.ds(j * GCHUNK, GCHUNK)
        gathers.append(pltpu.async_copy(
            table_hbm.at[idx_v.at[sl]], rows_v.at[sl, :], sems.at[j]))
    mask_copies = [
        pltpu.async_copy(mask_hbm.at[idx_v.at[pl.ds(j * GCHUNK, GCHUNK)]],
                         mval_v.at[pl.ds(j * GCHUNK, GCHUNK)], sem_mask)
        for j in range(NCHUNK)
    ]
    writes = []
    for j in range(NCHUNK):
        sl = pl.ds(j * GCHUNK, GCHUNK)
        gathers[j].wait()
        writes.append(pltpu.async_copy(
            rows_v.at[sl, :], rows_out.at[pl.ds(base + j * GCHUNK, GCHUNK)],
            sem_out))
    for c in mask_copies:
        c.wait()
    pltpu.sync_copy(mval_v, mask_out.at[pl.ds(base, B_PER_W)])
    for c in writes:
        c.wait()


def _sc_gather(idx, table, leaf_mask):
    mesh = plsc.VectorSubcoreMesh(core_axis_name="c", subcore_axis_name="s")
    k = pl.kernel(
        _sc_gather_body,
        out_type=(
            jax.ShapeDtypeStruct((BATCH, D), jnp.float32),
            jax.ShapeDtypeStruct((BATCH,), jnp.int32),
        ),
        mesh=mesh,
        scratch_types=[
            pltpu.VMEM((B_PER_W,), jnp.int32),
            pltpu.VMEM((B_PER_W, D), jnp.float32),
            pltpu.VMEM((B_PER_W,), jnp.int32),
            pltpu.SemaphoreType.DMA((NCHUNK,)),
            pltpu.SemaphoreType.DMA,
            pltpu.SemaphoreType.DMA,
        ],
    )
    return k(idx, table, leaf_mask)


ROWS_BLK = 4096


_DN_RT = (((0,), (1,)), ((), ()))
_DN_LT = (((0,), (0,)), ((), ()))


def _tc_mlp_body(x_ref, m_ref, w10_ref, b10_ref, w20_ref, b20_ref,
                 w11_ref, b11_ref, w21_ref, b21_ref, o0_ref, o1_ref):
    x = x_ref[...]
    nl = (m_ref[...] == 0)[None, :]
    e0 = x[:, :RANK]
    e1 = x[:, RANK:]
    eye = (lax.broadcasted_iota(jnp.int32, (RANK, RANK), 0)
           == lax.broadcasted_iota(jnp.int32, (RANK, RANK), 1)
           ).astype(jnp.float32)
    e0t = lax.dot_general(eye, e0, _DN_RT,
                          preferred_element_type=jnp.float32)
    e1t = lax.dot_general(eye, e1, _DN_RT,
                          preferred_element_type=jnp.float32)
    h0t = jnp.maximum(
        lax.dot_general(w10_ref[...], e0, _DN_RT,
                        preferred_element_type=jnp.float32) + b10_ref[...],
        0.0)
    o0t = lax.dot_general(w20_ref[...], h0t, _DN_LT,
                          preferred_element_type=jnp.float32) + b20_ref[...]
    h1t = jnp.maximum(
        lax.dot_general(w11_ref[...], e1, _DN_RT,
                        preferred_element_type=jnp.float32) + b11_ref[...],
        0.0)
    o1t = lax.dot_general(w21_ref[...], h1t, _DN_LT,
                          preferred_element_type=jnp.float32) + b21_ref[...]
    o0_ref[...] = jnp.where(nl, o0t, e0t)
    o1_ref[...] = jnp.where(nl, o1t, e1t)


def _tc_mlp(rows, mvals, w1t0, b1c0, w2t0, b2c0, w1t1, b1c1, w2t1, b2c1):
    grid = (BATCH // ROWS_BLK,)
    wspec = pl.BlockSpec((RANK, RANK), lambda i: (0, 0))
    bspec = pl.BlockSpec((RANK, 1), lambda i: (0, 0))
    return pl.pallas_call(
        _tc_mlp_body,
        grid=grid,
        in_specs=[
            pl.BlockSpec((ROWS_BLK, D), lambda i: (i, 0)),
            pl.BlockSpec((ROWS_BLK,), lambda i: (i,)),
            wspec, bspec, wspec, bspec,
            wspec, bspec, wspec, bspec,
        ],
        out_specs=[
            pl.BlockSpec((RANK, ROWS_BLK), lambda i: (0, i)),
            pl.BlockSpec((RANK, ROWS_BLK), lambda i: (0, i)),
        ],
        out_shape=[
            jax.ShapeDtypeStruct((RANK, BATCH), jnp.float32),
            jax.ShapeDtypeStruct((RANK, BATCH), jnp.float32),
        ],
    )(rows, mvals, w1t0, b1c0, w2t0, b2c0, w1t1, b1c1, w2t1, b2c1)


def kernel(nodeIdx, leaf_mask, table,
           W1_0, b1_0, W2_0, b2_0, W1_1, b1_1, W2_1, b2_1):
    idx = nodeIdx.astype(jnp.int32)
    rows, mvals = _sc_gather(idx, table, leaf_mask.astype(jnp.int32))
    o0t, o1t = _tc_mlp(
        rows, mvals,
        W1_0, b1_0.reshape(RANK, 1), W2_0, b2_0.reshape(RANK, 1),
        W1_1, b1_1.reshape(RANK, 1), W2_1, b2_1.reshape(RANK, 1))
    return (o0t.T, o1t.T)
